# baseline (speedup 1.0000x reference)
import functools

import jax
import jax.numpy as jnp
from jax import lax
from jax.experimental import pallas as pl
from jax.experimental.pallas import tpu as pltpu
from jax.experimental.pallas import tpu_sc as plsc

N = 10000
E = 160000
HID = 256
K = 5000
BLK = 2000
GRID = N // BLK

NC = 2
NS = 16
NW = NC * NS
RB = 320
NPAD = NW * RB
FLUSH = 2048
CAP = 82 * FLUSH
CH = 2000
EB = 80


def _row_spec(d=HID):
    return pl.BlockSpec((BLK, d), lambda i: (i, 0))


def _full_spec(shape):
    return pl.BlockSpec(shape, lambda i: tuple(0 for _ in shape))


def _mm_relu_body(x_ref, w_ref, b_ref, out_ref):
    out_ref[...] = jax.nn.relu(
        jnp.dot(x_ref[...], w_ref[...], preferred_element_type=jnp.float32)
        + b_ref[...][0][None, :])


def mm_relu(x, w, b):
    return pl.pallas_call(
        _mm_relu_body,
        grid=(GRID,),
        in_specs=[_row_spec(), _full_spec((HID, HID)), _full_spec((1, HID))],
        out_specs=_row_spec(),
        out_shape=jax.ShapeDtypeStruct((N, HID), jnp.float32),
    )(x, w, b.reshape(1, HID))


def _combine_body(feat_ref, neigh_ref, sw_ref, nw_ref, b_ref,
                  raw_ref, stat_ref):
    raw = (jnp.dot(feat_ref[...], sw_ref[...], preferred_element_type=jnp.float32)
           + jnp.dot(neigh_ref[...], nw_ref[...], preferred_element_type=jnp.float32)
           + b_ref[...][0][None, :])
    raw_ref[...] = raw
    s1 = jnp.sum(raw, axis=0, keepdims=True)
    s2 = jnp.sum(raw * raw, axis=0, keepdims=True)
    upd = jnp.concatenate(
        [s1, s2, jnp.zeros((6, HID), jnp.float32)], axis=0)

    @pl.when(pl.program_id(0) == 0)
    def _():
        stat_ref[...] = upd

    @pl.when(pl.program_id(0) > 0)
    def _():
        stat_ref[...] += upd


def combine_stats(feat, neigh, self_W, neigh_W, bias):
    return pl.pallas_call(
        _combine_body,
        grid=(GRID,),
        in_specs=[_row_spec(), _row_spec(), _full_spec((HID, HID)),
                  _full_spec((HID, HID)), _full_spec((1, HID))],
        out_specs=(_row_spec(), _full_spec((8, HID))),
        out_shape=(jax.ShapeDtypeStruct((N, HID), jnp.float32),
                   jax.ShapeDtypeStruct((8, HID), jnp.float32)),
    )(feat, neigh, self_W, neigh_W, bias.reshape(1, HID))


def _bn(raw, stat_ref, misc_ref):
    mu = stat_ref[0][None, :] * (1.0 / N)
    var = stat_ref[1][None, :] * (1.0 / N) - mu * mu
    feat = misc_ref[0][None, :] * (raw - mu) * jax.lax.rsqrt(var + 1e-5) \
        + misc_ref[1][None, :]
    return jax.nn.relu(feat)


def _norm_pool_body(raw_ref, stat_ref, misc_ref, pw_ref, feat_ref, hp_ref):
    feat = _bn(raw_ref[...], stat_ref, misc_ref)
    feat_ref[...] = feat
    hp_ref[...] = jax.nn.relu(
        jnp.dot(feat, pw_ref[...], preferred_element_type=jnp.float32)
        + misc_ref[2][None, :])


def norm_pool(raw, stats, bn_g, bn_b, pool_W, pool_b):
    misc = jnp.stack([bn_g, bn_b, pool_b], axis=0)
    return pl.pallas_call(
        _norm_pool_body,
        grid=(GRID,),
        in_specs=[_row_spec(), _full_spec((8, HID)), _full_spec((3, HID)),
                  _full_spec((HID, HID))],
        out_specs=(_row_spec(), _row_spec()),
        out_shape=(jax.ShapeDtypeStruct((N, HID), jnp.float32),
                   jax.ShapeDtypeStruct((N, HID), jnp.float32)),
    )(raw, stats, misc, pool_W)


def _norm_scale_body(raw_ref, stat_ref, misc_ref, deg_ref, feat_ref, hs_ref):
    feat = _bn(raw_ref[...], stat_ref, misc_ref)
    feat_ref[...] = feat
    hs_ref[...] = feat * jax.lax.rsqrt(jnp.maximum(deg_ref[...], 1.0))


def norm_scale(raw, stats, bn_g, bn_b, deg_out):
    misc = jnp.stack([bn_g, bn_b], axis=0)
    return pl.pallas_call(
        _norm_scale_body,
        grid=(GRID,),
        in_specs=[_row_spec(), _full_spec((8, HID)), _full_spec((2, HID)),
                  pl.BlockSpec((BLK, 1), lambda i: (i, 0))],
        out_specs=(_row_spec(), _row_spec()),
        out_shape=(jax.ShapeDtypeStruct((N, HID), jnp.float32),
                   jax.ShapeDtypeStruct((N, HID), jnp.float32)),
    )(raw, stats, misc, deg_out.reshape(N, 1))


def _select_body(agg_ref, deg_ref, sagw_ref, sagb_ref, w_ref, sel_ref):
    agg = agg_ref[0:N, :] * jax.lax.rsqrt(jnp.maximum(deg_ref[...], 1.0))
    score = (jnp.dot(agg, sagw_ref[...], preferred_element_type=jnp.float32)
             + sagb_ref[0, 0])
    bits = jax.lax.bitcast_convert_type(score, jnp.int32)
    mask = jnp.where(bits < 0, jnp.uint32(0xFFFFFFFF), jnp.uint32(0x80000000))
    key = jax.lax.bitcast_convert_type(bits, jnp.uint32) ^ mask

    def tbody(i, t):
        cand = t | (jnp.uint32(1) << (jnp.uint32(31) - jnp.uint32(i)))
        cnt = jnp.sum(jnp.where(key >= cand, 1, 0))
        return jnp.where(cnt >= K, cand, t)

    thr = jax.lax.fori_loop(0, 32, tbody, jnp.uint32(0))
    gt = key > thr
    tie = key == thr
    need = K - jnp.sum(jnp.where(gt, 1, 0))
    idx = jax.lax.broadcasted_iota(jnp.int32, score.shape, 0)

    def pbody(i, p):
        cand = p | (jnp.int32(1) << (jnp.int32(13) - i))
        cnt = jnp.sum(jnp.where(tie & (idx < cand), 1, 0))
        return jnp.where(cnt <= need, cand, p)

    pcut = jax.lax.fori_loop(0, 14, pbody, jnp.int32(0))
    sel = gt | (tie & (idx < pcut))
    w_ref[...] = jnp.where(sel, jnp.tanh(score), 0.0)
    sel_ref[...] = jnp.where(sel, 1.0, 0.0)


def select(agg, deg_in, sag_W, sag_b):
    return pl.pallas_call(
        _select_body,
        out_shape=(jax.ShapeDtypeStruct((N, 1), jnp.float32),
                   jax.ShapeDtypeStruct((N, 1), jnp.float32)),
    )(agg, deg_in.reshape(N, 1), sag_W, sag_b.reshape(1, 1))


def _readout_body(feat_ref, w_ref, sel_ref, sum_ref, max_ref):
    contrib = feat_ref[...] * w_ref[...]
    s = jnp.sum(contrib, axis=0, keepdims=True)
    m = jnp.max(jnp.where(sel_ref[...] > 0.0, contrib, -jnp.inf),
                axis=0, keepdims=True)
    s8 = jnp.concatenate([s, jnp.zeros((7, HID), jnp.float32)], axis=0)
    m8 = jnp.concatenate([m, jnp.full((7, HID), -jnp.inf, jnp.float32)],
                         axis=0)

    @pl.when(pl.program_id(0) == 0)
    def _():
        sum_ref[...] = s8
        max_ref[...] = m8

    @pl.when(pl.program_id(0) > 0)
    def _():
        sum_ref[...] += s8
        max_ref[...] = jnp.maximum(max_ref[...], m8)


def readout(feat, w, sel):
    return pl.pallas_call(
        _readout_body,
        grid=(GRID,),
        in_specs=[_row_spec(), pl.BlockSpec((BLK, 1), lambda i: (i, 0)),
                  pl.BlockSpec((BLK, 1), lambda i: (i, 0))],
        out_specs=(_full_spec((8, HID)), _full_spec((8, HID))),
        out_shape=(jax.ShapeDtypeStruct((8, HID), jnp.float32),
                   jax.ShapeDtypeStruct((8, HID), jnp.float32)),
    )(feat, w, sel)


def _mlp_body(sum_ref, max_ref, l1w_ref, l1b_ref, l2w_ref, l2b_ref,
              l3w_ref, l3b_ref, out_ref):
    avg = sum_ref[0][None, :] * (1.0 / K)
    mx = max_ref[0][None, :]
    h = jnp.concatenate([avg, mx], axis=1)
    h = jax.nn.relu(jnp.dot(h, l1w_ref[...], preferred_element_type=jnp.float32)
                    + l1b_ref[...][0][None, :])
    h = jax.nn.relu(jnp.dot(h, l2w_ref[...], preferred_element_type=jnp.float32)
                    + l2b_ref[...][0][None, :])
    logits = (jnp.dot(h, l3w_ref[...], preferred_element_type=jnp.float32)
              + l3b_ref[...][0][None, :])
    m = jnp.max(logits, axis=1, keepdims=True)
    lse = jnp.log(jnp.sum(jnp.exp(logits - m), axis=1, keepdims=True)) + m
    out_ref[...] = logits - lse


def mlp(sums, maxs, lin1_W, lin1_b, lin2_W, lin2_b, lin3_W, lin3_b):
    return pl.pallas_call(
        _mlp_body,
        out_shape=jax.ShapeDtypeStruct((1, 10), jnp.float32),
    )(sums, maxs, lin1_W, lin1_b.reshape(1, HID), lin2_W,
      lin2_b.reshape(1, HID // 2), lin3_W, lin3_b.reshape(1, 10))


_MESH = plsc.VectorSubcoreMesh(core_axis_name="c", subcore_axis_name="s")


@functools.partial(
    pl.kernel,
    mesh=_MESH,
    out_type=(jax.ShapeDtypeStruct((NW * CAP,), jnp.int32),
              jax.ShapeDtypeStruct((NW * 16,), jnp.int32),
              jax.ShapeDtypeStruct((NPAD,), jnp.float32),
              jax.ShapeDtypeStruct((NPAD,), jnp.float32)),
    compiler_params=pltpu.CompilerParams(needs_layout_passes=False),
    scratch_types=[
        pltpu.VMEM((CH,), jnp.int32),
        pltpu.VMEM((CH,), jnp.int32),
        pltpu.VMEM((CH,), jnp.int32),
        pltpu.VMEM((CH,), jnp.int32),
        pltpu.SemaphoreType.DMA,
        pltpu.SemaphoreType.DMA,
        pltpu.SemaphoreType.DMA,
        pltpu.SemaphoreType.DMA,
        pltpu.VMEM((2 * FLUSH,), jnp.int32),
        pltpu.VMEM((16 * RB,), jnp.int32),
        pltpu.VMEM((16 * RB,), jnp.int32),
        pltpu.VMEM((RB,), jnp.float32),
        pltpu.VMEM((16,), jnp.int32),
    ],
)
def _preprocess(esrc_hbm, edst_hbm, packed_hbm, counts_hbm,
                degin_hbm, degout_hbm,
                srcka, dstka, srckb, dstkb, sas, sad, sbs, sbd,
                bufp, hin, hout, degstage, cntstage):
    w = lax.axis_index("s") * NC + lax.axis_index("c")
    lo = w * RB
    zero16 = jnp.zeros((16,), jnp.int32)

    def zh(i, _):
        hin[pl.ds(i * 16, 16)] = zero16
        hout[pl.ds(i * 16, 16)] = zero16
        return 0

    lax.fori_loop(0, 16 * RB // 16, zh, 0)

    def zb(i, _):
        bufp[pl.ds(i * 16, 16)] = zero16
        return 0

    lax.fori_loop(0, 2 * FLUSH // 16, zb, 0)

    iota16 = lax.iota(jnp.int32, 16)
    lanes = iota16 * RB
    full15 = jnp.full((16,), 15, jnp.int32)

    def start_chunk(k, srck, dstk, ss, sd):
        pltpu.async_copy(esrc_hbm.at[pl.ds(k * CH, CH)], srck, ss)
        pltpu.async_copy(edst_hbm.at[pl.ds(k * CH, CH)], dstk, sd)

    def wait_chunk(k, srck, dstk, ss, sd):
        pltpu.make_async_copy(esrc_hbm.at[pl.ds(k * CH, CH)], srck, ss).wait()
        pltpu.make_async_copy(edst_hbm.at[pl.ds(k * CH, CH)], dstk, sd).wait()

    def do_chunk(k, carry, srck, dstk):
        cur, fl = carry
        curv0 = jnp.full((16,), cur, jnp.int32)

        def vec_body(j, curv):
            vs = srck[pl.ds(j * 16, 16)]
            vd = dstk[pl.ds(j * 16, 16)]
            dl = vd - lo
            m = (vd >= lo) & (dl < RB)
            mi = jnp.where(m, 1, 0)
            hidx = lanes + dl
            hv = plsc.load_gather(hin, [hidx], mask=m)
            plsc.store_scatter(hin, [hidx], hv + 1, mask=m)
            sl = vs - lo
            ms = (vs >= lo) & (sl < RB)
            hidx2 = lanes + sl
            hv2 = plsc.load_gather(hout, [hidx2], mask=ms)
            plsc.store_scatter(hout, [hidx2], hv2 + 1, mask=ms)
            tot = curv + plsc.cumsum(mi)
            pos = tot - 1
            packv = (vs << 9) | dl
            plsc.store_scatter(bufp, [pos], packv, mask=m)
            return jnp.take_along_axis(tot, full15, axis=0,
                                       mode="promise_in_bounds")

        curv = lax.fori_loop(0, CH // 16, vec_body, curv0)
        cur = curv[0]

        def do_flush(args):
            cur, fl = args
            pltpu.sync_copy(bufp.at[pl.ds(0, FLUSH)],
                            packed_hbm.at[pl.ds(w * CAP + fl * FLUSH, FLUSH)])
            for i in range(FLUSH // 16):
                ts = bufp[pl.ds(FLUSH + i * 16, 16)]
                bufp[pl.ds(i * 16, 16)] = ts
            return cur - FLUSH, fl + 1

        return lax.cond(cur >= FLUSH, do_flush, lambda a: a, (cur, fl))

    NCHUNK = E // CH
    LASTC = NCHUNK - 1

    def chunk_pair(t, carry):
        k0 = 2 * t
        start_chunk(jnp.minimum(k0 + 1, LASTC), srckb, dstkb, sbs, sbd)
        wait_chunk(k0, srcka, dstka, sas, sad)
        carry = do_chunk(k0, carry, srcka, dstka)
        start_chunk(jnp.minimum(k0 + 2, LASTC), srcka, dstka, sas, sad)
        wait_chunk(k0 + 1, srckb, dstkb, sbs, sbd)
        carry = do_chunk(k0 + 1, carry, srckb, dstkb)
        return carry

    start_chunk(jnp.int32(0), srcka, dstka, sas, sad)
    cur, fl = lax.fori_loop(0, NCHUNK // 2, chunk_pair,
                            (jnp.int32(0), jnp.int32(0)))
    pltpu.make_async_copy(esrc_hbm.at[pl.ds(0, CH)], srcka, sas).wait()
    pltpu.make_async_copy(edst_hbm.at[pl.ds(0, CH)], dstka, sad).wait()
    curv = jnp.full((16,), cur, jnp.int32)
    sent = jnp.full((16,), RB, jnp.int32)
    for i in range(FLUSH // 16):
        gi = iota16 + (i * 16)
        v = bufp[pl.ds(i * 16, 16)]
        bufp[pl.ds(i * 16, 16)] = jnp.where(gi >= curv, sent, v)
    pltpu.sync_copy(bufp.at[pl.ds(0, FLUSH)],
                    packed_hbm.at[pl.ds(w * CAP + fl * FLUSH, FLUSH)])
    for i in range(FLUSH // 16):
        bufp[pl.ds(i * 16, 16)] = sent
    pltpu.sync_copy(bufp.at[pl.ds(0, FLUSH)],
                    packed_hbm.at[pl.ds(w * CAP + (fl + 1) * FLUSH, FLUSH)])
    cntstage[...] = jnp.full((16,), fl * FLUSH + cur, jnp.int32)
    pltpu.sync_copy(cntstage, counts_hbm.at[pl.ds(w * 16, 16)])
    for dhbm, hist in ((degin_hbm, hin), (degout_hbm, hout)):
        for t in range(RB // 16):
            accv = jnp.zeros((16,), jnp.float32)
            for lq in range(16):
                accv = accv + hist[pl.ds(lq * RB + t * 16, 16)].astype(
                    jnp.float32)
            degstage[pl.ds(t * 16, 16)] = accv
        pltpu.sync_copy(degstage, dhbm.at[pl.ds(lo, RB)])


def _make_segment(is_max):
    @functools.partial(
        pl.kernel,
        mesh=_MESH,
        out_type=jax.ShapeDtypeStruct((NPAD, HID), jnp.float32),
        compiler_params=pltpu.CompilerParams(needs_layout_passes=False),
        scratch_types=[
            pltpu.VMEM((RB + 1, HID), jnp.float32),
            pltpu.VMEM((EB, HID), jnp.float32),
            pltpu.VMEM((EB, HID), jnp.float32),
            pltpu.VMEM((EB,), jnp.int32),
            pltpu.VMEM((EB,), jnp.int32),
            pltpu.VMEM((EB,), jnp.int32),
            pltpu.VMEM((EB,), jnp.int32),
            pltpu.VMEM((EB,), jnp.int32),
            pltpu.VMEM((EB,), jnp.int32),
            pltpu.VMEM((16,), jnp.int32),
            pltpu.SemaphoreType.DMA,
            pltpu.SemaphoreType.DMA,
            pltpu.SemaphoreType.DMA,
            pltpu.SemaphoreType.DMA,
        ],
    )
    def seg(h_hbm, packed_hbm, counts_hbm, out_hbm,
            acc, rows0, rows1, pk0, pk1, idxv0, idxv1, dlv0, dlv1,
            cntv, sem0, sem1, p0, p1):
        w = lax.axis_index("s") * NC + lax.axis_index("c")
        zero16f = jnp.zeros((16,), jnp.float32)

        def zacc(i, _):
            for c in range(16):
                acc[i, pl.ds(c * 16, 16)] = zero16f
            return 0

        lax.fori_loop(0, RB + 1, zacc, 0)
        pltpu.sync_copy(counts_hbm.at[pl.ds(w * 16, 16)], cntv)
        count = jnp.max(cntv[...])
        nb = (count + (EB - 1)) // EB

        def pk_start(b, pk, sp):
            pltpu.async_copy(packed_hbm.at[pl.ds(w * CAP + b * EB, EB)],
                             pk, sp)

        def pk_wait(pk, sp):
            pltpu.make_async_copy(packed_hbm.at[pl.ds(w * CAP, EB)],
                                  pk, sp).wait()

        def build(pk, idxv, dlv):
            for q in range(EB // 16):
                v = pk[pl.ds(q * 16, 16)]
                idxv[pl.ds(q * 16, 16)] = lax.shift_right_logical(v, 9)
                dlv[pl.ds(q * 16, 16)] = v & 511

        def process(dlv, rows):
            def group(g, _):
                chunk = dlv[pl.ds(g * 16, 16)]
                for lane in range(16):
                    d = chunk[lane]
                    e = g * 16 + lane
                    hs = [rows[e, pl.ds(c * 16, 16)] for c in range(16)]
                    avs = [acc[d, pl.ds(c * 16, 16)] for c in range(16)]
                    if is_max:
                        res = [jnp.maximum(a, h) for a, h in zip(avs, hs)]
                    else:
                        res = [a + h for a, h in zip(avs, hs)]
                    for c in range(16):
                        acc[d, pl.ds(c * 16, 16)] = res[c]
                return 0

            lax.fori_loop(0, EB // 16, group, 0)

        pltpu.sync_copy(packed_hbm.at[pl.ds(w * CAP, EB)], pk0)
        build(pk0, idxv0, dlv0)
        pltpu.async_copy(h_hbm.at[idxv0], rows0, sem0)
        pk_start(jnp.int32(1), pk1, p1)
        npair = (nb + 1) // 2

        def pair(t, _):
            b0 = 2 * t
            pk_wait(pk1, p1)
            build(pk1, idxv1, dlv1)
            pltpu.async_copy(h_hbm.at[idxv1], rows1, sem1)
            pk_start(b0 + 2, pk0, p0)
            pltpu.make_async_copy(h_hbm.at[idxv0], rows0, sem0).wait()
            process(dlv0, rows0)
            pk_wait(pk0, p0)
            build(pk0, idxv0, dlv0)
            pltpu.async_copy(h_hbm.at[idxv0], rows0, sem0)
            pk_start(b0 + 3, pk1, p1)
            pltpu.make_async_copy(h_hbm.at[idxv1], rows1, sem1).wait()
            process(dlv1, rows1)
            return 0

        lax.fori_loop(0, npair, pair, 0)
        pltpu.make_async_copy(h_hbm.at[idxv0], rows0, sem0).wait()
        pk_wait(pk1, p1)
        pltpu.sync_copy(acc.at[pl.ds(0, RB)], out_hbm.at[pl.ds(w * RB, RB)])

    return seg


_segment_max_sc = _make_segment(True)
_segment_sum_sc = _make_segment(False)


def kernel(x, edge_index, pool_W0, pool_b0, self_W0, neigh_W0, bias0, bn_g0, bn_b0, pool_W1, pool_b1, self_W1, neigh_W1, bias1, bn_g1, bn_b1, pool_W2, pool_b2, self_W2, neigh_W2, bias2, bn_g2, bn_b2, pool_W3, pool_b3, self_W3, neigh_W3, bias3, bn_g3, bn_b3, sag_W, sag_b, lin1_W, lin1_b, lin2_W, lin2_b, lin3_W, lin3_b):
    params = [
        (self_W0, neigh_W0, bias0, bn_g0, bn_b0),
        (self_W1, neigh_W1, bias1, bn_g1, bn_b1),
        (self_W2, neigh_W2, bias2, bn_g2, bn_b2),
        (self_W3, neigh_W3, bias3, bn_g3, bn_b3),
    ]
    pools = [(pool_W1, pool_b1), (pool_W2, pool_b2), (pool_W3, pool_b3)]

    packed, counts, deg_in_pad, deg_out_pad = _preprocess(
        edge_index[0], edge_index[1])
    deg_in = deg_in_pad[:N]
    deg_out = deg_out_pad[:N]

    feat = x
    hp = mm_relu(x, pool_W0, pool_b0)
    for i in range(4):
        neigh = _segment_max_sc(hp, packed, counts)
        sw, nw, b, g, bb = params[i]
        raw, stats = combine_stats(feat, neigh, sw, nw, b)
        if i < 3:
            pw, pb = pools[i]
            feat, hp = norm_pool(raw, stats, g, bb, pw, pb)
        else:
            feat, hs = norm_scale(raw, stats, g, bb, deg_out)
    agg = _segment_sum_sc(hs, packed, counts)
    w, sel = select(agg, deg_in, sag_W, sag_b)
    sums, maxs = readout(feat, w, sel)
    return mlp(sums, maxs, lin1_W, lin1_b, lin2_W, lin2_b, lin3_W, lin3_b)


# speedup vs baseline: 1.9354x; 1.9354x over previous
import functools

import jax
import jax.numpy as jnp
from jax import lax
from jax.experimental import pallas as pl
from jax.experimental.pallas import tpu as pltpu
from jax.experimental.pallas import tpu_sc as plsc

N = 10000
E = 160000
HID = 256
K = 5000
BLK = 2000
GRID = N // BLK

NC = 2
NS = 16
NW = NC * NS
RB = 320
NPAD = NW * RB
FLUSH = 2048
CAP = 82 * FLUSH
CH = 2000
EB = 80


def _row_spec(d=HID):
    return pl.BlockSpec((BLK, d), lambda i: (i, 0))


def _full_spec(shape):
    return pl.BlockSpec(shape, lambda i: tuple(0 for _ in shape))


def _mm_relu_body(x_ref, w_ref, b_ref, out_ref):
    out_ref[...] = jax.nn.relu(
        jnp.dot(x_ref[...], w_ref[...], preferred_element_type=jnp.float32)
        + b_ref[...][0][None, :])


def mm_relu(x, w, b):
    return pl.pallas_call(
        _mm_relu_body,
        grid=(GRID,),
        in_specs=[_row_spec(), _full_spec((HID, HID)), _full_spec((1, HID))],
        out_specs=_row_spec(),
        out_shape=jax.ShapeDtypeStruct((N, HID), jnp.float32),
    )(x, w, b.reshape(1, HID))


def _combine_body(feat_ref, neigh_ref, sw_ref, nw_ref, b_ref,
                  raw_ref, stat_ref):
    raw = (jnp.dot(feat_ref[...], sw_ref[...], preferred_element_type=jnp.float32)
           + jnp.dot(neigh_ref[...], nw_ref[...], preferred_element_type=jnp.float32)
           + b_ref[...][0][None, :])
    raw_ref[...] = raw
    s1 = jnp.sum(raw, axis=0, keepdims=True)
    s2 = jnp.sum(raw * raw, axis=0, keepdims=True)
    upd = jnp.concatenate(
        [s1, s2, jnp.zeros((6, HID), jnp.float32)], axis=0)

    @pl.when(pl.program_id(0) == 0)
    def _():
        stat_ref[...] = upd

    @pl.when(pl.program_id(0) > 0)
    def _():
        stat_ref[...] += upd


def combine_stats(feat, neigh, self_W, neigh_W, bias):
    return pl.pallas_call(
        _combine_body,
        grid=(GRID,),
        in_specs=[_row_spec(), _row_spec(), _full_spec((HID, HID)),
                  _full_spec((HID, HID)), _full_spec((1, HID))],
        out_specs=(_row_spec(), _full_spec((8, HID))),
        out_shape=(jax.ShapeDtypeStruct((N, HID), jnp.float32),
                   jax.ShapeDtypeStruct((8, HID), jnp.float32)),
    )(feat, neigh, self_W, neigh_W, bias.reshape(1, HID))


def _bn(raw, stat_ref, misc_ref):
    mu = stat_ref[0][None, :] * (1.0 / N)
    var = stat_ref[1][None, :] * (1.0 / N) - mu * mu
    feat = misc_ref[0][None, :] * (raw - mu) * jax.lax.rsqrt(var + 1e-5) \
        + misc_ref[1][None, :]
    return jax.nn.relu(feat)


def _norm_pool_body(raw_ref, stat_ref, misc_ref, pw_ref, feat_ref, hp_ref):
    feat = _bn(raw_ref[...], stat_ref, misc_ref)
    feat_ref[...] = feat
    hp_ref[...] = jax.nn.relu(
        jnp.dot(feat, pw_ref[...], preferred_element_type=jnp.float32)
        + misc_ref[2][None, :])


def norm_pool(raw, stats, bn_g, bn_b, pool_W, pool_b):
    misc = jnp.stack([bn_g, bn_b, pool_b], axis=0)
    return pl.pallas_call(
        _norm_pool_body,
        grid=(GRID,),
        in_specs=[_row_spec(), _full_spec((8, HID)), _full_spec((3, HID)),
                  _full_spec((HID, HID))],
        out_specs=(_row_spec(), _row_spec()),
        out_shape=(jax.ShapeDtypeStruct((N, HID), jnp.float32),
                   jax.ShapeDtypeStruct((N, HID), jnp.float32)),
    )(raw, stats, misc, pool_W)


def _norm_scale_body(raw_ref, stat_ref, misc_ref, deg_ref, feat_ref, hs_ref):
    feat = _bn(raw_ref[...], stat_ref, misc_ref)
    feat_ref[...] = feat
    hs_ref[...] = feat * jax.lax.rsqrt(jnp.maximum(deg_ref[...], 1.0))


def norm_scale(raw, stats, bn_g, bn_b, deg_out):
    misc = jnp.stack([bn_g, bn_b], axis=0)
    return pl.pallas_call(
        _norm_scale_body,
        grid=(GRID,),
        in_specs=[_row_spec(), _full_spec((8, HID)), _full_spec((2, HID)),
                  pl.BlockSpec((BLK, 1), lambda i: (i, 0))],
        out_specs=(_row_spec(), _row_spec()),
        out_shape=(jax.ShapeDtypeStruct((N, HID), jnp.float32),
                   jax.ShapeDtypeStruct((N, HID), jnp.float32)),
    )(raw, stats, misc, deg_out.reshape(N, 1))


def _select_body(agg_ref, deg_ref, sagw_ref, sagb_ref, w_ref, sel_ref):
    agg = agg_ref[0:N, :] * jax.lax.rsqrt(jnp.maximum(deg_ref[...], 1.0))
    score = (jnp.dot(agg, sagw_ref[...], preferred_element_type=jnp.float32)
             + sagb_ref[0, 0])
    bits = jax.lax.bitcast_convert_type(score, jnp.int32)
    mask = jnp.where(bits < 0, jnp.uint32(0xFFFFFFFF), jnp.uint32(0x80000000))
    key = jax.lax.bitcast_convert_type(bits, jnp.uint32) ^ mask

    def tbody(i, t):
        cand = t | (jnp.uint32(1) << (jnp.uint32(31) - jnp.uint32(i)))
        cnt = jnp.sum(jnp.where(key >= cand, 1, 0))
        return jnp.where(cnt >= K, cand, t)

    thr = jax.lax.fori_loop(0, 32, tbody, jnp.uint32(0))
    gt = key > thr
    tie = key == thr
    need = K - jnp.sum(jnp.where(gt, 1, 0))
    idx = jax.lax.broadcasted_iota(jnp.int32, score.shape, 0)

    def pbody(i, p):
        cand = p | (jnp.int32(1) << (jnp.int32(13) - i))
        cnt = jnp.sum(jnp.where(tie & (idx < cand), 1, 0))
        return jnp.where(cnt <= need, cand, p)

    pcut = jax.lax.fori_loop(0, 14, pbody, jnp.int32(0))
    sel = gt | (tie & (idx < pcut))
    w_ref[...] = jnp.where(sel, jnp.tanh(score), 0.0)
    sel_ref[...] = jnp.where(sel, 1.0, 0.0)


def select(agg, deg_in, sag_W, sag_b):
    return pl.pallas_call(
        _select_body,
        out_shape=(jax.ShapeDtypeStruct((N, 1), jnp.float32),
                   jax.ShapeDtypeStruct((N, 1), jnp.float32)),
    )(agg, deg_in.reshape(N, 1), sag_W, sag_b.reshape(1, 1))


def _readout_body(feat_ref, w_ref, sel_ref, sum_ref, max_ref):
    contrib = feat_ref[...] * w_ref[...]
    s = jnp.sum(contrib, axis=0, keepdims=True)
    m = jnp.max(jnp.where(sel_ref[...] > 0.0, contrib, -jnp.inf),
                axis=0, keepdims=True)
    s8 = jnp.concatenate([s, jnp.zeros((7, HID), jnp.float32)], axis=0)
    m8 = jnp.concatenate([m, jnp.full((7, HID), -jnp.inf, jnp.float32)],
                         axis=0)

    @pl.when(pl.program_id(0) == 0)
    def _():
        sum_ref[...] = s8
        max_ref[...] = m8

    @pl.when(pl.program_id(0) > 0)
    def _():
        sum_ref[...] += s8
        max_ref[...] = jnp.maximum(max_ref[...], m8)


def readout(feat, w, sel):
    return pl.pallas_call(
        _readout_body,
        grid=(GRID,),
        in_specs=[_row_spec(), pl.BlockSpec((BLK, 1), lambda i: (i, 0)),
                  pl.BlockSpec((BLK, 1), lambda i: (i, 0))],
        out_specs=(_full_spec((8, HID)), _full_spec((8, HID))),
        out_shape=(jax.ShapeDtypeStruct((8, HID), jnp.float32),
                   jax.ShapeDtypeStruct((8, HID), jnp.float32)),
    )(feat, w, sel)


def _mlp_body(sum_ref, max_ref, l1w_ref, l1b_ref, l2w_ref, l2b_ref,
              l3w_ref, l3b_ref, out_ref):
    avg = sum_ref[0][None, :] * (1.0 / K)
    mx = max_ref[0][None, :]
    h = jnp.concatenate([avg, mx], axis=1)
    h = jax.nn.relu(jnp.dot(h, l1w_ref[...], preferred_element_type=jnp.float32)
                    + l1b_ref[...][0][None, :])
    h = jax.nn.relu(jnp.dot(h, l2w_ref[...], preferred_element_type=jnp.float32)
                    + l2b_ref[...][0][None, :])
    logits = (jnp.dot(h, l3w_ref[...], preferred_element_type=jnp.float32)
              + l3b_ref[...][0][None, :])
    m = jnp.max(logits, axis=1, keepdims=True)
    lse = jnp.log(jnp.sum(jnp.exp(logits - m), axis=1, keepdims=True)) + m
    out_ref[...] = logits - lse


def mlp(sums, maxs, lin1_W, lin1_b, lin2_W, lin2_b, lin3_W, lin3_b):
    return pl.pallas_call(
        _mlp_body,
        out_shape=jax.ShapeDtypeStruct((1, 10), jnp.float32),
    )(sums, maxs, lin1_W, lin1_b.reshape(1, HID), lin2_W,
      lin2_b.reshape(1, HID // 2), lin3_W, lin3_b.reshape(1, 10))


_MESH = plsc.VectorSubcoreMesh(core_axis_name="c", subcore_axis_name="s")


@functools.partial(
    pl.kernel,
    mesh=_MESH,
    out_type=(jax.ShapeDtypeStruct((NW * CAP,), jnp.int32),
              jax.ShapeDtypeStruct((NW * 16,), jnp.int32),
              jax.ShapeDtypeStruct((NPAD,), jnp.float32),
              jax.ShapeDtypeStruct((NPAD,), jnp.float32)),
    compiler_params=pltpu.CompilerParams(needs_layout_passes=False),
    scratch_types=[
        pltpu.VMEM((CH,), jnp.int32),
        pltpu.VMEM((CH,), jnp.int32),
        pltpu.VMEM((CH,), jnp.int32),
        pltpu.VMEM((CH,), jnp.int32),
        pltpu.SemaphoreType.DMA,
        pltpu.SemaphoreType.DMA,
        pltpu.SemaphoreType.DMA,
        pltpu.SemaphoreType.DMA,
        pltpu.VMEM((2 * FLUSH,), jnp.int32),
        pltpu.VMEM((16 * RB,), jnp.int32),
        pltpu.VMEM((16 * RB,), jnp.int32),
        pltpu.VMEM((RB,), jnp.float32),
        pltpu.VMEM((16,), jnp.int32),
    ],
)
def _preprocess(esrc_hbm, edst_hbm, packed_hbm, counts_hbm,
                degin_hbm, degout_hbm,
                srcka, dstka, srckb, dstkb, sas, sad, sbs, sbd,
                bufp, hin, hout, degstage, cntstage):
    w = lax.axis_index("s") * NC + lax.axis_index("c")
    lo = w * RB
    zero16 = jnp.zeros((16,), jnp.int32)

    def zh(i, _):
        hin[pl.ds(i * 16, 16)] = zero16
        hout[pl.ds(i * 16, 16)] = zero16
        return 0

    lax.fori_loop(0, 16 * RB // 16, zh, 0)

    def zb(i, _):
        bufp[pl.ds(i * 16, 16)] = zero16
        return 0

    lax.fori_loop(0, 2 * FLUSH // 16, zb, 0)

    iota16 = lax.iota(jnp.int32, 16)
    lanes = iota16 * RB
    full15 = jnp.full((16,), 15, jnp.int32)

    def start_chunk(k, srck, dstk, ss, sd):
        pltpu.async_copy(esrc_hbm.at[pl.ds(k * CH, CH)], srck, ss)
        pltpu.async_copy(edst_hbm.at[pl.ds(k * CH, CH)], dstk, sd)

    def wait_chunk(k, srck, dstk, ss, sd):
        pltpu.make_async_copy(esrc_hbm.at[pl.ds(k * CH, CH)], srck, ss).wait()
        pltpu.make_async_copy(edst_hbm.at[pl.ds(k * CH, CH)], dstk, sd).wait()

    def do_chunk(k, carry, srck, dstk):
        cur, fl = carry
        curv0 = jnp.full((16,), cur, jnp.int32)

        def vec_body(j, curv):
            vs = srck[pl.ds(j * 16, 16)]
            vd = dstk[pl.ds(j * 16, 16)]
            dl = vd - lo
            m = (vd >= lo) & (dl < RB)
            mi = jnp.where(m, 1, 0)
            hidx = lanes + dl
            hv = plsc.load_gather(hin, [hidx], mask=m)
            plsc.store_scatter(hin, [hidx], hv + 1, mask=m)
            sl = vs - lo
            ms = (vs >= lo) & (sl < RB)
            hidx2 = lanes + sl
            hv2 = plsc.load_gather(hout, [hidx2], mask=ms)
            plsc.store_scatter(hout, [hidx2], hv2 + 1, mask=ms)
            tot = curv + plsc.cumsum(mi)
            pos = tot - 1
            packv = (vs << 9) | dl
            plsc.store_scatter(bufp, [pos], packv, mask=m)
            return jnp.take_along_axis(tot, full15, axis=0,
                                       mode="promise_in_bounds")

        curv = lax.fori_loop(0, CH // 16, vec_body, curv0)
        cur = curv[0]

        def do_flush(args):
            cur, fl = args
            pltpu.sync_copy(bufp.at[pl.ds(0, FLUSH)],
                            packed_hbm.at[pl.ds(w * CAP + fl * FLUSH, FLUSH)])
            for i in range(FLUSH // 16):
                ts = bufp[pl.ds(FLUSH + i * 16, 16)]
                bufp[pl.ds(i * 16, 16)] = ts
            return cur - FLUSH, fl + 1

        return lax.cond(cur >= FLUSH, do_flush, lambda a: a, (cur, fl))

    NCHUNK = E // CH
    LASTC = NCHUNK - 1

    def chunk_pair(t, carry):
        k0 = 2 * t
        start_chunk(jnp.minimum(k0 + 1, LASTC), srckb, dstkb, sbs, sbd)
        wait_chunk(k0, srcka, dstka, sas, sad)
        carry = do_chunk(k0, carry, srcka, dstka)
        start_chunk(jnp.minimum(k0 + 2, LASTC), srcka, dstka, sas, sad)
        wait_chunk(k0 + 1, srckb, dstkb, sbs, sbd)
        carry = do_chunk(k0 + 1, carry, srckb, dstkb)
        return carry

    start_chunk(jnp.int32(0), srcka, dstka, sas, sad)
    cur, fl = lax.fori_loop(0, NCHUNK // 2, chunk_pair,
                            (jnp.int32(0), jnp.int32(0)))
    pltpu.make_async_copy(esrc_hbm.at[pl.ds(0, CH)], srcka, sas).wait()
    pltpu.make_async_copy(edst_hbm.at[pl.ds(0, CH)], dstka, sad).wait()
    curv = jnp.full((16,), cur, jnp.int32)
    for i in range(FLUSH // 16):
        gi = iota16 + (i * 16)
        sent = ((gi + w * FLUSH // 16) << 9) | RB
        v = bufp[pl.ds(i * 16, 16)]
        bufp[pl.ds(i * 16, 16)] = jnp.where(gi >= curv, sent, v)
    pltpu.sync_copy(bufp.at[pl.ds(0, FLUSH)],
                    packed_hbm.at[pl.ds(w * CAP + fl * FLUSH, FLUSH)])
    for i in range(FLUSH // 16):
        gi = iota16 + (i * 16)
        bufp[pl.ds(i * 16, 16)] = ((gi + w * FLUSH // 16) << 9) | RB
    pltpu.sync_copy(bufp.at[pl.ds(0, FLUSH)],
                    packed_hbm.at[pl.ds(w * CAP + (fl + 1) * FLUSH, FLUSH)])
    cntstage[...] = jnp.full((16,), fl * FLUSH + cur, jnp.int32)
    pltpu.sync_copy(cntstage, counts_hbm.at[pl.ds(w * 16, 16)])
    for dhbm, hist in ((degin_hbm, hin), (degout_hbm, hout)):
        for t in range(RB // 16):
            accv = jnp.zeros((16,), jnp.float32)
            for lq in range(16):
                accv = accv + hist[pl.ds(lq * RB + t * 16, 16)].astype(
                    jnp.float32)
            degstage[pl.ds(t * 16, 16)] = accv
        pltpu.sync_copy(degstage, dhbm.at[pl.ds(lo, RB)])


def _make_segment(is_max):
    @functools.partial(
        pl.kernel,
        mesh=_MESH,
        out_type=jax.ShapeDtypeStruct((NPAD, HID), jnp.float32),
        compiler_params=pltpu.CompilerParams(needs_layout_passes=False),
        scratch_types=[
            pltpu.VMEM((RB + 1, HID), jnp.float32),
            pltpu.VMEM((EB, HID), jnp.float32),
            pltpu.VMEM((EB, HID), jnp.float32),
            pltpu.VMEM((EB,), jnp.int32),
            pltpu.VMEM((EB,), jnp.int32),
            pltpu.VMEM((EB,), jnp.int32),
            pltpu.VMEM((EB,), jnp.int32),
            pltpu.VMEM((EB,), jnp.int32),
            pltpu.VMEM((EB,), jnp.int32),
            pltpu.VMEM((16,), jnp.int32),
            pltpu.SemaphoreType.DMA,
            pltpu.SemaphoreType.DMA,
            pltpu.SemaphoreType.DMA,
            pltpu.SemaphoreType.DMA,
        ],
    )
    def seg(h_hbm, packed_hbm, counts_hbm, out_hbm,
            acc, rows0, rows1, pk0, pk1, idxv0, idxv1, dlv0, dlv1,
            cntv, sem0, sem1, p0, p1):
        w = lax.axis_index("s") * NC + lax.axis_index("c")
        zero16f = jnp.zeros((16,), jnp.float32)

        def zacc(i, _):
            for c in range(16):
                acc[i, pl.ds(c * 16, 16)] = zero16f
            return 0

        lax.fori_loop(0, RB + 1, zacc, 0)
        pltpu.sync_copy(counts_hbm.at[pl.ds(w * 16, 16)], cntv)
        count = jnp.max(cntv[...])
        nb = (count + (EB - 1)) // EB

        def pk_start(b, pk, sp):
            pltpu.async_copy(packed_hbm.at[pl.ds(w * CAP + b * EB, EB)],
                             pk, sp)

        def pk_wait(pk, sp):
            pltpu.make_async_copy(packed_hbm.at[pl.ds(w * CAP, EB)],
                                  pk, sp).wait()

        def build(pk, idxv, dlv):
            for q in range(EB // 16):
                v = pk[pl.ds(q * 16, 16)]
                idxv[pl.ds(q * 16, 16)] = lax.shift_right_logical(v, 9)
                dlv[pl.ds(q * 16, 16)] = v & 511

        def process(dlv, rows):
            def group(g, _):
                chunk = dlv[pl.ds(g * 16, 16)]
                for lane in range(16):
                    d = chunk[lane]
                    e = g * 16 + lane
                    hs = [rows[e, pl.ds(c * 16, 16)] for c in range(16)]
                    avs = [acc[d, pl.ds(c * 16, 16)] for c in range(16)]
                    if is_max:
                        res = [jnp.maximum(a, h) for a, h in zip(avs, hs)]
                    else:
                        res = [a + h for a, h in zip(avs, hs)]
                    for c in range(16):
                        acc[d, pl.ds(c * 16, 16)] = res[c]
                return 0

            lax.fori_loop(0, EB // 16, group, 0)

        pltpu.sync_copy(packed_hbm.at[pl.ds(w * CAP, EB)], pk0)
        build(pk0, idxv0, dlv0)
        pltpu.async_copy(h_hbm.at[idxv0], rows0, sem0)
        pk_start(jnp.int32(1), pk1, p1)
        npair = (nb + 1) // 2

        def pair(t, _):
            b0 = 2 * t
            pk_wait(pk1, p1)
            build(pk1, idxv1, dlv1)
            pltpu.async_copy(h_hbm.at[idxv1], rows1, sem1)
            pk_start(b0 + 2, pk0, p0)
            pltpu.make_async_copy(h_hbm.at[idxv0], rows0, sem0).wait()
            process(dlv0, rows0)
            pk_wait(pk0, p0)
            build(pk0, idxv0, dlv0)
            pltpu.async_copy(h_hbm.at[idxv0], rows0, sem0)
            pk_start(b0 + 3, pk1, p1)
            pltpu.make_async_copy(h_hbm.at[idxv1], rows1, sem1).wait()
            process(dlv1, rows1)
            return 0

        lax.fori_loop(0, npair, pair, 0)
        pltpu.make_async_copy(h_hbm.at[idxv0], rows0, sem0).wait()
        pk_wait(pk1, p1)
        pltpu.sync_copy(acc.at[pl.ds(0, RB)], out_hbm.at[pl.ds(w * RB, RB)])

    return seg


_segment_max_sc = _make_segment(True)
_segment_sum_sc = _make_segment(False)


def kernel(x, edge_index, pool_W0, pool_b0, self_W0, neigh_W0, bias0, bn_g0, bn_b0, pool_W1, pool_b1, self_W1, neigh_W1, bias1, bn_g1, bn_b1, pool_W2, pool_b2, self_W2, neigh_W2, bias2, bn_g2, bn_b2, pool_W3, pool_b3, self_W3, neigh_W3, bias3, bn_g3, bn_b3, sag_W, sag_b, lin1_W, lin1_b, lin2_W, lin2_b, lin3_W, lin3_b):
    params = [
        (self_W0, neigh_W0, bias0, bn_g0, bn_b0),
        (self_W1, neigh_W1, bias1, bn_g1, bn_b1),
        (self_W2, neigh_W2, bias2, bn_g2, bn_b2),
        (self_W3, neigh_W3, bias3, bn_g3, bn_b3),
    ]
    pools = [(pool_W1, pool_b1), (pool_W2, pool_b2), (pool_W3, pool_b3)]

    packed, counts, deg_in_pad, deg_out_pad = _preprocess(
        edge_index[0], edge_index[1])
    deg_in = deg_in_pad[:N]
    deg_out = deg_out_pad[:N]

    feat = x
    hp = mm_relu(x, pool_W0, pool_b0)
    for i in range(4):
        neigh = _segment_max_sc(hp, packed, counts)
        sw, nw, b, g, bb = params[i]
        raw, stats = combine_stats(feat, neigh, sw, nw, b)
        if i < 3:
            pw, pb = pools[i]
            feat, hp = norm_pool(raw, stats, g, bb, pw, pb)
        else:
            feat, hs = norm_scale(raw, stats, g, bb, deg_out)
    agg = _segment_sum_sc(hs, packed, counts)
    w, sel = select(agg, deg_in, sag_W, sag_b)
    sums, maxs = readout(feat, w, sel)
    return mlp(sums, maxs, lin1_W, lin1_b, lin2_W, lin2_b, lin3_W, lin3_b)
